# baseline (speedup 1.0000x reference)
import functools

import jax
import jax.numpy as jnp
from jax import lax
from jax.experimental import pallas as pl
from jax.experimental.pallas import tpu as pltpu
from jax.experimental.pallas import tpu_sc as plsc

B, NA, NBH = 8, 1024, 32
NB_ATOM, NF, NS = 128, 128, 25
CUTOFF = 0.8
LOG2 = 0.6931471805599453

E = B * NA * NBH
NW = 32
EPW = E // NW
CH = 128
NCH = EPW // CH
NF2 = NF // 2

TA = 128
NBLK = (B * NA) // TA


def _ssp(t):
    return jax.nn.softplus(t) - LOG2


def _in2f_kernel(x_ref, w_ref, o_ref):
    o_ref[...] = jnp.dot(x_ref[...], w_ref[...], preferred_element_type=jnp.float32)


def _in2f(x2, Wi):
    return pl.pallas_call(
        _in2f_kernel,
        grid=(B,),
        in_specs=[
            pl.BlockSpec((NA, NB_ATOM), lambda i: (i, 0)),
            pl.BlockSpec((NB_ATOM, NF), lambda i: (0, 0)),
        ],
        out_specs=pl.BlockSpec((NA, NF), lambda i: (i, 0)),
        out_shape=jax.ShapeDtypeStruct((B * NA, NF), jnp.float32),
    )(x2, Wi)


def _sc_gather(y2, idx1, n_edges):
    epw = n_edges // NW
    nch = epw // CH
    mesh = plsc.VectorSubcoreMesh(core_axis_name="c", subcore_axis_name="s")

    @functools.partial(
        pl.kernel,
        out_type=jax.ShapeDtypeStruct((n_edges, NF), jnp.float32),
        mesh=mesh,
        scratch_types=[
            pltpu.VMEM((epw,), jnp.int32),
            pltpu.VMEM((CH, NF), jnp.float32),
            pltpu.VMEM((CH, NF), jnp.float32),
            pltpu.SemaphoreType.DMA,
            pltpu.SemaphoreType.DMA,
        ],
    )
    def gather_k(y_hbm, idx_hbm, out_hbm, idx_v, rows0, rows1, sem0, sem1):
        wid = lax.axis_index("s") * 2 + lax.axis_index("c")
        base = wid * epw
        pltpu.sync_copy(idx_hbm.at[pl.ds(base, epw)], idx_v)

        pltpu.async_copy(y_hbm.at[idx_v.at[pl.ds(0, CH)]], rows0, sem0)

        def body(j, _):
            rows_cur = j % 2

            @pl.when(j + 1 < nch)
            def _():
                @pl.when(rows_cur == 0)
                def _():
                    pltpu.async_copy(
                        y_hbm.at[idx_v.at[pl.ds((j + 1) * CH, CH)]], rows1, sem1)

                @pl.when(rows_cur == 1)
                def _():
                    pltpu.async_copy(
                        y_hbm.at[idx_v.at[pl.ds((j + 1) * CH, CH)]], rows0, sem0)

            @pl.when(rows_cur == 0)
            def _():
                pltpu.make_async_copy(
                    y_hbm.at[idx_v.at[pl.ds(j * CH, CH)]], rows0, sem0).wait()
                pltpu.sync_copy(rows0, out_hbm.at[pl.ds(base + j * CH, CH)])

            @pl.when(rows_cur == 1)
            def _():
                pltpu.make_async_copy(
                    y_hbm.at[idx_v.at[pl.ds(j * CH, CH)]], rows1, sem1).wait()
                pltpu.sync_copy(rows1, out_hbm.at[pl.ds(base + j * CH, CH)])

            return 0

        lax.fori_loop(0, nch, body, 0)

    return gather_k(y2, idx1)


def _fused_kernel(f_ref, r_ref, m_ref, g_ref, w1, b1r, w2, b2r, wf, bfr, wd, bdr, o_ref):
    fb = f_ref[...].reshape(TA * NBH, NS)
    h = _ssp(jnp.dot(fb, w1[...], preferred_element_type=jnp.float32) + b1r[...])
    filt = jnp.dot(h, w2[...], preferred_element_type=jnp.float32) + b2r[...]
    c = jnp.where(r_ref[...] <= CUTOFF, 1.0, 0.0) * m_ref[...]
    gv = g_ref[...]
    prod = filt.reshape(TA, NBH, NF) * gv.reshape(TA, NBH, NF)
    t = jnp.sum(prod * c[:, :, None], axis=1)
    u = _ssp(jnp.dot(t, wf[...], preferred_element_type=jnp.float32) + bfr[...])
    o_ref[...] = jnp.dot(u, wd[...], preferred_element_type=jnp.float32) + bdr[...]


def _fused(f3, r2, m2, g2, W1, b1, W2, b2, Wf, bf, Wd, bd, blk0, nblk):
    const2 = lambda shape: pl.BlockSpec(shape, lambda i: (0, 0))
    return pl.pallas_call(
        _fused_kernel,
        grid=(nblk,),
        in_specs=[
            pl.BlockSpec((TA, NBH, NS), lambda i: (blk0 + i, 0, 0)),
            pl.BlockSpec((TA, NBH), lambda i: (blk0 + i, 0)),
            pl.BlockSpec((TA, NBH), lambda i: (blk0 + i, 0)),
            pl.BlockSpec((TA * NBH, NF), lambda i: (i, 0)),
            const2((NS, NF)),
            const2((1, NF)),
            const2((NF, NF)),
            const2((1, NF)),
            const2((NF, NB_ATOM)),
            const2((1, NB_ATOM)),
            const2((NB_ATOM, NB_ATOM)),
            const2((1, NB_ATOM)),
        ],
        out_specs=pl.BlockSpec((TA, NB_ATOM), lambda i: (i, 0)),
        out_shape=jax.ShapeDtypeStruct((nblk * TA, NB_ATOM), jnp.float32),
    )(f3, r2, m2, g2, W1, b1, W2, b2, Wf, bf, Wd, bd)


def kernel(x, r_ij, neighbors, neighbor_mask, f_ij, W1, b1, W2, b2, Wi, Wf, bf, Wd, bd):
    x2 = x.reshape(B * NA, NB_ATOM)
    y2 = _in2f(x2, Wi)

    nb = neighbors.astype(jnp.int32)
    idx = jnp.arange(B, dtype=jnp.int32)[:, None, None] * NA + nb
    idx1 = idx.reshape(E)

    f3 = f_ij.reshape(B * NA, NBH, NS)
    r2 = r_ij.reshape(B * NA, NBH)
    m2 = neighbor_mask.reshape(B * NA, NBH)
    b1r, b2r = b1.reshape(1, NF), b2.reshape(1, NF)
    bfr, bdr = bf.reshape(1, NB_ATOM), bd.reshape(1, NB_ATOM)

    ns = 4
    eh = E // ns
    nblk_h = NBLK // ns
    outs = []
    gs = [
        _sc_gather(y2, idx1[h * eh:(h + 1) * eh], eh)
        for h in range(ns)
    ]
    for h in range(ns):
        outs.append(_fused(
            f3, r2, m2, gs[h],
            W1, b1r, W2, b2r, Wf, bfr, Wd, bdr,
            h * nblk_h, nblk_h,
        ))
    out = jnp.concatenate(outs, axis=0)
    return out.reshape(B, NA, NB_ATOM)


# speedup vs baseline: 1.0032x; 1.0032x over previous
import functools

import jax
import jax.numpy as jnp
from jax import lax
from jax.experimental import pallas as pl
from jax.experimental.pallas import tpu as pltpu
from jax.experimental.pallas import tpu_sc as plsc

B, NA, NBH = 8, 1024, 32
NB_ATOM, NF, NS = 128, 128, 25
CUTOFF = 0.8
LOG2 = 0.6931471805599453

E = B * NA * NBH
NW = 32
EPW = E // NW
CH = 128
NCH = EPW // CH
NF2 = NF // 2

TA = 128
NBLK = (B * NA) // TA


def _ssp(t):
    return jax.nn.softplus(t) - LOG2


def _in2f_kernel(x_ref, w_ref, o_ref):
    o_ref[...] = jnp.dot(x_ref[...], w_ref[...], preferred_element_type=jnp.float32)


def _in2f(x2, Wi):
    return pl.pallas_call(
        _in2f_kernel,
        grid=(B,),
        in_specs=[
            pl.BlockSpec((NA, NB_ATOM), lambda i: (i, 0)),
            pl.BlockSpec((NB_ATOM, NF), lambda i: (0, 0)),
        ],
        out_specs=pl.BlockSpec((NA, NF), lambda i: (i, 0)),
        out_shape=jax.ShapeDtypeStruct((B * NA, NF), jnp.float32),
    )(x2, Wi)


def _sc_gather(y2, idx1, n_edges):
    epw = n_edges // NW
    nch = epw // CH
    mesh = plsc.VectorSubcoreMesh(core_axis_name="c", subcore_axis_name="s")

    @functools.partial(
        pl.kernel,
        out_type=jax.ShapeDtypeStruct((n_edges, NF), jnp.float32),
        mesh=mesh,
        scratch_types=[
            pltpu.VMEM((epw,), jnp.int32),
            pltpu.VMEM((CH, NF), jnp.float32),
            pltpu.VMEM((CH, NF), jnp.float32),
            pltpu.SemaphoreType.DMA,
            pltpu.SemaphoreType.DMA,
        ],
    )
    def gather_k(y_hbm, idx_hbm, out_hbm, idx_v, rows0, rows1, sem0, sem1):
        wid = lax.axis_index("s") * 2 + lax.axis_index("c")
        base = wid * epw
        pltpu.sync_copy(idx_hbm.at[pl.ds(base, epw)], idx_v)

        pltpu.async_copy(y_hbm.at[idx_v.at[pl.ds(0, CH)]], rows0, sem0)

        def body(j, _):
            rows_cur = j % 2

            @pl.when(j + 1 < nch)
            def _():
                @pl.when(rows_cur == 0)
                def _():
                    pltpu.async_copy(
                        y_hbm.at[idx_v.at[pl.ds((j + 1) * CH, CH)]], rows1, sem1)

                @pl.when(rows_cur == 1)
                def _():
                    pltpu.async_copy(
                        y_hbm.at[idx_v.at[pl.ds((j + 1) * CH, CH)]], rows0, sem0)

            @pl.when(rows_cur == 0)
            def _():
                pltpu.make_async_copy(
                    y_hbm.at[idx_v.at[pl.ds(j * CH, CH)]], rows0, sem0).wait()
                pltpu.sync_copy(rows0, out_hbm.at[pl.ds(base + j * CH, CH)])

            @pl.when(rows_cur == 1)
            def _():
                pltpu.make_async_copy(
                    y_hbm.at[idx_v.at[pl.ds(j * CH, CH)]], rows1, sem1).wait()
                pltpu.sync_copy(rows1, out_hbm.at[pl.ds(base + j * CH, CH)])

            return 0

        lax.fori_loop(0, nch, body, 0)

    return gather_k(y2, idx1)


def _fused_kernel(f_ref, r_ref, m_ref, g_ref, w1, b1r, w2, b2r, wf, bfr, wd, bdr, o_ref):
    fb = f_ref[...].reshape(TA * NBH, NS)
    h = _ssp(jnp.dot(fb, w1[...], preferred_element_type=jnp.float32) + b1r[...])
    filt = jnp.dot(h, w2[...], preferred_element_type=jnp.float32) + b2r[...]
    c = jnp.where(r_ref[...] <= CUTOFF, 1.0, 0.0) * m_ref[...]
    gv = g_ref[...]
    prod = filt.reshape(TA, NBH, NF) * gv.reshape(TA, NBH, NF)
    t = jnp.sum(prod * c[:, :, None], axis=1)
    u = _ssp(jnp.dot(t, wf[...], preferred_element_type=jnp.float32) + bfr[...])
    o_ref[...] = jnp.dot(u, wd[...], preferred_element_type=jnp.float32) + bdr[...]


def _fused(f3, r2, m2, g2, W1, b1, W2, b2, Wf, bf, Wd, bd, blk0, nblk):
    const2 = lambda shape: pl.BlockSpec(shape, lambda i: (0, 0))
    return pl.pallas_call(
        _fused_kernel,
        grid=(nblk,),
        in_specs=[
            pl.BlockSpec((TA, NBH, NS), lambda i: (blk0 + i, 0, 0)),
            pl.BlockSpec((TA, NBH), lambda i: (blk0 + i, 0)),
            pl.BlockSpec((TA, NBH), lambda i: (blk0 + i, 0)),
            pl.BlockSpec((TA * NBH, NF), lambda i: (i, 0)),
            const2((NS, NF)),
            const2((1, NF)),
            const2((NF, NF)),
            const2((1, NF)),
            const2((NF, NB_ATOM)),
            const2((1, NB_ATOM)),
            const2((NB_ATOM, NB_ATOM)),
            const2((1, NB_ATOM)),
        ],
        out_specs=pl.BlockSpec((TA, NB_ATOM), lambda i: (i, 0)),
        out_shape=jax.ShapeDtypeStruct((nblk * TA, NB_ATOM), jnp.float32),
    )(f3, r2, m2, g2, W1, b1, W2, b2, Wf, bf, Wd, bd)


def kernel(x, r_ij, neighbors, neighbor_mask, f_ij, W1, b1, W2, b2, Wi, Wf, bf, Wd, bd):
    x2 = x.reshape(B * NA, NB_ATOM)
    y2 = _in2f(x2, Wi)

    nb = neighbors.astype(jnp.int32)
    idx = jnp.arange(B, dtype=jnp.int32)[:, None, None] * NA + nb
    idx1 = idx.reshape(E)

    f3 = f_ij.reshape(B * NA, NBH, NS)
    r2 = r_ij.reshape(B * NA, NBH)
    m2 = neighbor_mask.reshape(B * NA, NBH)
    b1r, b2r = b1.reshape(1, NF), b2.reshape(1, NF)
    bfr, bdr = bf.reshape(1, NB_ATOM), bd.reshape(1, NB_ATOM)

    ns = 1
    eh = E // ns
    nblk_h = NBLK // ns
    outs = []
    gs = [
        _sc_gather(y2, idx1[h * eh:(h + 1) * eh], eh)
        for h in range(ns)
    ]
    for h in range(ns):
        outs.append(_fused(
            f3, r2, m2, gs[h],
            W1, b1r, W2, b2r, Wf, bfr, Wd, bdr,
            h * nblk_h, nblk_h,
        ))
    out = jnp.concatenate(outs, axis=0)
    return out.reshape(B, NA, NB_ATOM)


# speedup vs baseline: 1.0221x; 1.0188x over previous
import functools

import jax
import jax.numpy as jnp
from jax import lax
from jax.experimental import pallas as pl
from jax.experimental.pallas import tpu as pltpu
from jax.experimental.pallas import tpu_sc as plsc

B, NA, NBH = 8, 1024, 32
NB_ATOM, NF, NS = 128, 128, 25
CUTOFF = 0.8
LOG2 = 0.6931471805599453

E = B * NA * NBH
NW = 32
EPW = E // NW
CH = 128
NCH = EPW // CH
NF2 = NF // 2

TA = 128
NBLK = (B * NA) // TA


def _ssp(t):
    return jax.nn.softplus(t) - LOG2


def _in2f_kernel(x_ref, w_ref, o_ref):
    o_ref[...] = jnp.dot(x_ref[...], w_ref[...], preferred_element_type=jnp.float32)


def _in2f(x2, Wi):
    return pl.pallas_call(
        _in2f_kernel,
        grid=(B,),
        in_specs=[
            pl.BlockSpec((NA, NB_ATOM), lambda i: (i, 0)),
            pl.BlockSpec((NB_ATOM, NF), lambda i: (0, 0)),
        ],
        out_specs=pl.BlockSpec((NA, NF), lambda i: (i, 0)),
        out_shape=jax.ShapeDtypeStruct((B * NA, NF), jnp.float32),
    )(x2, Wi)


def _sc_gather(y2, idx1, n_edges):
    epw = n_edges // NW
    nch = epw // CH
    mesh = plsc.VectorSubcoreMesh(core_axis_name="c", subcore_axis_name="s")

    @functools.partial(
        pl.kernel,
        out_type=jax.ShapeDtypeStruct((n_edges, NF), jnp.float32),
        mesh=mesh,
        scratch_types=[
            pltpu.VMEM((epw,), jnp.int32),
            pltpu.VMEM((CH, NF), jnp.float32),
            pltpu.VMEM((CH, NF), jnp.float32),
            pltpu.SemaphoreType.DMA,
            pltpu.SemaphoreType.DMA,
        ],
    )
    def gather_k(y_hbm, idx_hbm, out_hbm, idx_v, rows0, rows1, sem0, sem1):
        wid = lax.axis_index("s") * 2 + lax.axis_index("c")
        base = wid * epw
        pltpu.sync_copy(idx_hbm.at[pl.ds(base, epw)], idx_v)

        pltpu.async_copy(y_hbm.at[idx_v.at[pl.ds(0, CH)]], rows0, sem0)

        def body(j, _):
            rows_cur = j % 2

            @pl.when(j + 1 < nch)
            def _():
                @pl.when(rows_cur == 0)
                def _():
                    pltpu.async_copy(
                        y_hbm.at[idx_v.at[pl.ds((j + 1) * CH, CH)]], rows1, sem1)

                @pl.when(rows_cur == 1)
                def _():
                    pltpu.async_copy(
                        y_hbm.at[idx_v.at[pl.ds((j + 1) * CH, CH)]], rows0, sem0)

            @pl.when(rows_cur == 0)
            def _():
                pltpu.make_async_copy(
                    y_hbm.at[idx_v.at[pl.ds(j * CH, CH)]], rows0, sem0).wait()
                pltpu.sync_copy(rows0, out_hbm.at[pl.ds(base + j * CH, CH)])

            @pl.when(rows_cur == 1)
            def _():
                pltpu.make_async_copy(
                    y_hbm.at[idx_v.at[pl.ds(j * CH, CH)]], rows1, sem1).wait()
                pltpu.sync_copy(rows1, out_hbm.at[pl.ds(base + j * CH, CH)])

            return 0

        lax.fori_loop(0, nch, body, 0)

    return gather_k(y2, idx1)


def _fused_kernel(f_ref, r_ref, m_ref, g_ref, w1, b1r, w2, b2r, wf, bfr, wd, bdr, o_ref):
    fb = f_ref[...].reshape(TA * NBH, NS)
    h = _ssp(jnp.dot(fb, w1[...], preferred_element_type=jnp.float32) + b1r[...])
    filt = jnp.dot(h, w2[...], preferred_element_type=jnp.float32) + b2r[...]
    c = jnp.where(r_ref[...] <= CUTOFF, 1.0, 0.0) * m_ref[...]
    gv = g_ref[...]
    prod = filt.reshape(TA, NBH, NF) * gv.reshape(TA, NBH, NF)
    t = jnp.sum(prod * c[:, :, None], axis=1)
    u = _ssp(jnp.dot(t, wf[...], preferred_element_type=jnp.float32) + bfr[...])
    o_ref[...] = jnp.dot(u, wd[...], preferred_element_type=jnp.float32) + bdr[...]


def _fused(f3, r2, m2, g2, W1, b1, W2, b2, Wf, bf, Wd, bd, blk0, nblk):
    const2 = lambda shape: pl.BlockSpec(shape, lambda i: (0, 0))
    return pl.pallas_call(
        _fused_kernel,
        grid=(nblk,),
        in_specs=[
            pl.BlockSpec((TA, NBH, NS), lambda i: (blk0 + i, 0, 0)),
            pl.BlockSpec((TA, NBH), lambda i: (blk0 + i, 0)),
            pl.BlockSpec((TA, NBH), lambda i: (blk0 + i, 0)),
            pl.BlockSpec((TA * NBH, NF), lambda i: (i, 0)),
            const2((NS, NF)),
            const2((1, NF)),
            const2((NF, NF)),
            const2((1, NF)),
            const2((NF, NB_ATOM)),
            const2((1, NB_ATOM)),
            const2((NB_ATOM, NB_ATOM)),
            const2((1, NB_ATOM)),
        ],
        out_specs=pl.BlockSpec((TA, NB_ATOM), lambda i: (i, 0)),
        out_shape=jax.ShapeDtypeStruct((nblk * TA, NB_ATOM), jnp.float32),
    )(f3, r2, m2, g2, W1, b1, W2, b2, Wf, bf, Wd, bd)


def kernel(x, r_ij, neighbors, neighbor_mask, f_ij, W1, b1, W2, b2, Wi, Wf, bf, Wd, bd):
    x2 = x.reshape(B * NA, NB_ATOM)
    y2 = _in2f(x2, Wi)

    nb = neighbors.astype(jnp.int32)
    idx = jnp.arange(B, dtype=jnp.int32)[:, None, None] * NA + nb
    idx1 = idx.reshape(E)

    f3 = f_ij.reshape(B * NA, NBH, NS)
    r2 = r_ij.reshape(B * NA, NBH)
    m2 = neighbor_mask.reshape(B * NA, NBH)
    b1r, b2r = b1.reshape(1, NF), b2.reshape(1, NF)
    bfr, bdr = bf.reshape(1, NB_ATOM), bd.reshape(1, NB_ATOM)

    ns = 2
    eh = E // ns
    nblk_h = NBLK // ns
    outs = []
    gs = [
        _sc_gather(y2, idx1[h * eh:(h + 1) * eh], eh)
        for h in range(ns)
    ]
    for h in range(ns):
        outs.append(_fused(
            f3, r2, m2, gs[h],
            W1, b1r, W2, b2r, Wf, bfr, Wd, bdr,
            h * nblk_h, nblk_h,
        ))
    out = jnp.concatenate(outs, axis=0)
    return out.reshape(B, NA, NB_ATOM)


# speedup vs baseline: 1.0829x; 1.0595x over previous
import functools

import jax
import jax.numpy as jnp
from jax import lax
from jax.experimental import pallas as pl
from jax.experimental.pallas import tpu as pltpu
from jax.experimental.pallas import tpu_sc as plsc

B, NA, NBH = 8, 1024, 32
NB_ATOM, NF, NS = 128, 128, 25
CUTOFF = 0.8
LOG2 = 0.6931471805599453

E = B * NA * NBH
NW = 32
CH = 128

TA = 128
NAB = NA // TA


def _ssp(t):
    return jax.nn.softplus(t) - LOG2


def _in2f_kernel(x_ref, w_ref, o_ref):
    o_ref[...] = jnp.dot(x_ref[...], w_ref[...], preferred_element_type=jnp.float32)


def _in2f(x2, Wi):
    return pl.pallas_call(
        _in2f_kernel,
        grid=(B,),
        in_specs=[
            pl.BlockSpec((NA, NB_ATOM), lambda i: (i, 0)),
            pl.BlockSpec((NB_ATOM, NF), lambda i: (0, 0)),
        ],
        out_specs=pl.BlockSpec((NA, NF), lambda i: (i, 0)),
        out_shape=jax.ShapeDtypeStruct((B * NA, NF), jnp.float32),
    )(x2, Wi)


def _sc_gather(y2, idx1, n_edges):
    epw = n_edges // NW
    nch = epw // CH
    mesh = plsc.VectorSubcoreMesh(core_axis_name="c", subcore_axis_name="s")

    @functools.partial(
        pl.kernel,
        out_type=jax.ShapeDtypeStruct((n_edges, NF), jnp.float32),
        mesh=mesh,
        scratch_types=[
            pltpu.VMEM((epw,), jnp.int32),
            pltpu.VMEM((CH, NF), jnp.float32),
            pltpu.VMEM((CH, NF), jnp.float32),
            pltpu.SemaphoreType.DMA,
            pltpu.SemaphoreType.DMA,
        ],
    )
    def gather_k(y_hbm, idx_hbm, out_hbm, idx_v, rows0, rows1, sem0, sem1):
        wid = lax.axis_index("s") * 2 + lax.axis_index("c")
        base = wid * epw
        pltpu.sync_copy(idx_hbm.at[pl.ds(base, epw)], idx_v)

        pltpu.async_copy(y_hbm.at[idx_v.at[pl.ds(0, CH)]], rows0, sem0)

        def body(j, _):
            rows_cur = j % 2

            @pl.when(j + 1 < nch)
            def _():
                @pl.when(rows_cur == 0)
                def _():
                    pltpu.async_copy(
                        y_hbm.at[idx_v.at[pl.ds((j + 1) * CH, CH)]], rows1, sem1)

                @pl.when(rows_cur == 1)
                def _():
                    pltpu.async_copy(
                        y_hbm.at[idx_v.at[pl.ds((j + 1) * CH, CH)]], rows0, sem0)

            @pl.when(rows_cur == 0)
            def _():
                pltpu.make_async_copy(
                    y_hbm.at[idx_v.at[pl.ds(j * CH, CH)]], rows0, sem0).wait()
                pltpu.sync_copy(rows0, out_hbm.at[pl.ds(base + j * CH, CH)])

            @pl.when(rows_cur == 1)
            def _():
                pltpu.make_async_copy(
                    y_hbm.at[idx_v.at[pl.ds(j * CH, CH)]], rows1, sem1).wait()
                pltpu.sync_copy(rows1, out_hbm.at[pl.ds(base + j * CH, CH)])

            return 0

        lax.fori_loop(0, nch, body, 0)

    return gather_k(y2, idx1)


def _fused_kernel(ft_ref, r_ref, m_ref, g_ref, w1, b1r, w2, b2r, wf, bfr, wd, bdr, o_ref):
    c = jnp.where(r_ref[...] <= CUTOFF, 1.0, 0.0) * m_ref[...]
    t = jnp.zeros((TA, NF), jnp.float32)
    for j in range(NBH):
        ftj = ft_ref[0, :, j, :]
        hj = _ssp(
            lax.dot_general(ftj, w1[...], (((0,), (0,)), ((), ())),
                            preferred_element_type=jnp.float32) + b1r[...])
        filtj = jnp.dot(hj, w2[...], preferred_element_type=jnp.float32) + b2r[...]
        t = t + filtj * g_ref[0, j] * c[:, j:j + 1]
    u = _ssp(jnp.dot(t, wf[...], preferred_element_type=jnp.float32) + bfr[...])
    o_ref[0] = jnp.dot(u, wd[...], preferred_element_type=jnp.float32) + bdr[...]


def _fused(ft4, r2, m2, g4, W1, b1, W2, b2, Wf, bf, Wd, bd):
    const2 = lambda shape: pl.BlockSpec(shape, lambda b, i: (0, 0))
    return pl.pallas_call(
        _fused_kernel,
        grid=(B, NAB),
        in_specs=[
            pl.BlockSpec((1, NS, NBH, TA), lambda b, i: (b, 0, 0, i)),
            pl.BlockSpec((TA, NBH), lambda b, i: (b * NAB + i, 0)),
            pl.BlockSpec((TA, NBH), lambda b, i: (b * NAB + i, 0)),
            pl.BlockSpec((1, NBH, TA, NF), lambda b, i: (b, 0, i, 0)),
            const2((NS, NF)),
            const2((1, NF)),
            const2((NF, NF)),
            const2((1, NF)),
            const2((NF, NB_ATOM)),
            const2((1, NB_ATOM)),
            const2((NB_ATOM, NB_ATOM)),
            const2((1, NB_ATOM)),
        ],
        out_specs=pl.BlockSpec((1, TA, NB_ATOM), lambda b, i: (b, i, 0)),
        out_shape=jax.ShapeDtypeStruct((B, NA, NB_ATOM), jnp.float32),
    )(ft4, r2, m2, g4, W1, b1, W2, b2, Wf, bf, Wd, bd)


def kernel(x, r_ij, neighbors, neighbor_mask, f_ij, W1, b1, W2, b2, Wi, Wf, bf, Wd, bd):
    x2 = x.reshape(B * NA, NB_ATOM)
    y2 = _in2f(x2, Wi)

    nbT = jnp.transpose(neighbors.astype(jnp.int32), (0, 2, 1))
    idx = jnp.arange(B, dtype=jnp.int32)[:, None, None] * NA + nbT
    idx1 = idx.reshape(E)

    g = _sc_gather(y2, idx1, E)
    g4 = g.reshape(B, NBH, NA, NF)

    ft4 = jnp.transpose(f_ij, (0, 3, 2, 1))
    r2 = r_ij.reshape(B * NA, NBH)
    m2 = neighbor_mask.reshape(B * NA, NBH)
    b1r, b2r = b1.reshape(1, NF), b2.reshape(1, NF)
    bfr, bdr = bf.reshape(1, NB_ATOM), bd.reshape(1, NB_ATOM)

    return _fused(ft4, r2, m2, g4, W1, b1r, W2, b2r, Wf, bfr, Wd, bdr)
